# baseline (speedup 1.0000x reference)
import functools

import jax
import jax.numpy as jnp
from jax import lax
from jax.experimental import pallas as pl
from jax.experimental.pallas import tpu as pltpu
from jax.experimental.pallas import tpu_sc as plsc

N = 10000
D = 128
CHUNK = 128
NCHUNK = 80
NW = 32
CAP = NW * NCHUNK * CHUNK
ACC_ROWS = 10240
ZROWS = ACC_ROWS // 16
GROUP = 16
CH0 = 160
CH1 = 2 * NCHUNK - CH0
GS_ROWS = 10112
GZ = GS_ROWS // 16
RBLK = 2000

_mesh = plsc.VectorSubcoreMesh(core_axis_name="c", subcore_axis_name="s")


@functools.partial(
    pl.kernel,
    out_type=jax.ShapeDtypeStruct((2 * ACC_ROWS,), jnp.float32),
    mesh=_mesh,
    scratch_types=[
        pltpu.VMEM((NCHUNK, CHUNK), jnp.int32),
        pltpu.VMEM((2, CHUNK), jnp.float32),
        pltpu.VMEM_SHARED((ACC_ROWS,), jnp.float32),
    ],
)
def _cnt_call(dst_hbm, const_hbm, out_hbm, dst_v, const_v, acc):
    c = lax.axis_index("c")
    s = lax.axis_index("s")
    w = s * 2 + c
    pltpu.sync_copy(dst_hbm.at[pl.ds(w * NCHUNK, NCHUNK)], dst_v)
    pltpu.sync_copy(const_hbm, const_v)
    for j in range(ZROWS // CHUNK):
        pltpu.sync_copy(const_v.at[1], acc.at[pl.ds(s * ZROWS + j * CHUNK, CHUNK)])
    plsc.subcore_barrier()

    def body(i, _):
        pltpu.sync_copy(const_v.at[0], acc.at[dst_v.at[i]], add=True)
        return 0

    lax.fori_loop(0, NCHUNK, body, 0)
    plsc.subcore_barrier()
    pltpu.sync_copy(acc.at[pl.ds(s * ZROWS, ZROWS)],
                    out_hbm.at[pl.ds(c * ACC_ROWS + s * ZROWS, ZROWS)])


@functools.partial(
    pl.kernel,
    out_type=jax.ShapeDtypeStruct((2 * GS_ROWS, D), jnp.float32),
    mesh=_mesh,
    scratch_types=[
        pltpu.VMEM((2, GROUP, CHUNK), jnp.int32),
        pltpu.VMEM((2, GROUP, CHUNK), jnp.int32),
        pltpu.VMEM((CHUNK, D), jnp.float32),
        pltpu.VMEM((CHUNK, D), jnp.float32),
        pltpu.VMEM_SHARED((GS_ROWS, D), jnp.float32),
        pltpu.SemaphoreType.DMA,
        pltpu.SemaphoreType.DMA,
        pltpu.SemaphoreType.DMA,
    ],
)
def _gs_call(hp_hbm, src_hbm, dst_hbm, zeros_hbm, out_hbm,
             sidx, didx, r0, r1, acc, semi, sem0, sem1):
    c = lax.axis_index("c")
    s = lax.axis_index("s")

    pltpu.sync_copy(zeros_hbm, r0)
    for j in range(4):
        pltpu.sync_copy(r0, acc.at[pl.ds(s * GZ + j * CHUNK, CHUNK)])
    pltpu.sync_copy(r0.at[pl.ds(0, GZ - 4 * CHUNK)],
                    acc.at[pl.ds(s * GZ + 4 * CHUNK, GZ - 4 * CHUNK)])
    plsc.subcore_barrier()

    def pipeline(base, ngroup):
        if ngroup == 0:
            return
        def idx_copies(g):
            b = g % 2
            rows = pl.ds(base + g * GROUP, GROUP)
            return (pltpu.make_async_copy(src_hbm.at[rows], sidx.at[b], semi),
                    pltpu.make_async_copy(dst_hbm.at[rows], didx.at[b], semi))

        for cp in idx_copies(0):
            cp.start()
        for g in range(ngroup):
            for cp in idx_copies(g):
                cp.wait()
            if g + 1 < ngroup:
                for cp in idx_copies(g + 1):
                    cp.start()
            sv = sidx.at[g % 2]
            dv = didx.at[g % 2]
            pltpu.async_copy(hp_hbm.at[sv.at[0]], r0, sem0)
            pltpu.async_copy(hp_hbm.at[sv.at[1]], r1, sem1)

            def body(p, _, sv=sv, dv=dv):
                i = 2 * p
                pltpu.make_async_copy(hp_hbm.at[sv.at[i]], r0, sem0).wait()
                pltpu.sync_copy(r0, acc.at[dv.at[i]], add=True)
                pltpu.async_copy(hp_hbm.at[sv.at[i + 2]], r0, sem0)
                pltpu.make_async_copy(hp_hbm.at[sv.at[i + 1]], r1, sem1).wait()
                pltpu.sync_copy(r1, acc.at[dv.at[i + 1]], add=True)
                pltpu.async_copy(hp_hbm.at[sv.at[i + 3]], r1, sem1)
                return 0

            lax.fori_loop(0, GROUP // 2 - 1, body, 0)
            pltpu.make_async_copy(hp_hbm.at[sv.at[GROUP - 2]], r0, sem0).wait()
            pltpu.sync_copy(r0, acc.at[dv.at[GROUP - 2]], add=True)
            pltpu.make_async_copy(hp_hbm.at[sv.at[GROUP - 1]], r1, sem1).wait()
            pltpu.sync_copy(r1, acc.at[dv.at[GROUP - 1]], add=True)

    @pl.when(c == 0)
    def _():
        pipeline(s * CH0, CH0 // GROUP)

    @pl.when(c == 1)
    def _():
        pipeline(16 * CH0 + s * CH1, CH1 // GROUP)

    plsc.subcore_barrier()
    pltpu.sync_copy(acc.at[pl.ds(s * GZ, GZ)],
                    out_hbm.at[pl.ds(c * GS_ROWS + s * GZ, GZ)])


def _dis(cnt_ref):
    deg = cnt_ref[:, 0:1] + cnt_ref[:, 1:2] + 1.0
    return lax.rsqrt(deg)


def _dense1_body(cnt_ref, x_ref, w_ref, o_ref):
    xw = jnp.dot(x_ref[...], w_ref[...], preferred_element_type=jnp.float32,
                 precision=lax.Precision.HIGHEST)
    o_ref[...] = xw * _dis(cnt_ref)


def _dense2_body(cnt_ref, s0_ref, s1_ref, hp_ref, b_ref, w_ref, o_ref):
    dis = _dis(cnt_ref)
    h1 = jnp.maximum(dis * (s0_ref[...] + s1_ref[...] + hp_ref[...]) + b_ref[...],
                     0.0)
    o_ref[...] = dis * jnp.dot(h1, w_ref[...], preferred_element_type=jnp.float32,
                               precision=lax.Precision.HIGHEST)


def _dense3_body(cnt_ref, s0_ref, s1_ref, hp_ref, b_ref, h_ref, g_ref):
    dis = _dis(cnt_ref)
    h2 = dis * (s0_ref[...] + s1_ref[...] + hp_ref[...]) + b_ref[...]
    h_ref[...] = h2

    @pl.when(pl.program_id(0) == 0)
    def _():
        g_ref[...] = jnp.zeros_like(g_ref)

    g_ref[...] += jnp.sum(h2, axis=0, keepdims=True) * (1.0 / N)


_row_spec = pl.BlockSpec((RBLK, D), lambda i: (i, 0))
_cnt_spec = pl.BlockSpec((RBLK, 2), lambda i: (i, 0))
_full_spec = pl.BlockSpec((D, D), lambda i: (0, 0))
_b_spec = pl.BlockSpec((1, D), lambda i: (0, 0))

_dense1 = pl.pallas_call(
    _dense1_body, grid=(N // RBLK,),
    in_specs=[_cnt_spec, _row_spec, _full_spec],
    out_specs=_row_spec,
    out_shape=jax.ShapeDtypeStruct((N, D), jnp.float32))

_dense2 = pl.pallas_call(
    _dense2_body, grid=(N // RBLK,),
    in_specs=[_cnt_spec, _row_spec, _row_spec, _row_spec, _b_spec, _full_spec],
    out_specs=_row_spec,
    out_shape=jax.ShapeDtypeStruct((N, D), jnp.float32))

_dense3 = pl.pallas_call(
    _dense3_body, grid=(N // RBLK,),
    in_specs=[_cnt_spec, _row_spec, _row_spec, _row_spec, _b_spec],
    out_specs=[_row_spec, pl.BlockSpec((1, D), lambda i: (0, 0))],
    out_shape=[jax.ShapeDtypeStruct((N, D), jnp.float32),
               jax.ShapeDtypeStruct((1, D), jnp.float32)])


def kernel(x, edge_index, W1, b1, W2, b2):
    src = edge_index[0].astype(jnp.int32)
    dst = edge_index[1].astype(jnp.int32)
    e = src.shape[0]
    pad = CAP - e
    src_p = jnp.concatenate([src, jnp.zeros((pad,), jnp.int32)]
                            ).reshape(NW * NCHUNK, CHUNK)
    dst_p = jnp.concatenate([dst, jnp.full((pad,), N, jnp.int32)]
                            ).reshape(NW * NCHUNK, CHUNK)
    const = jnp.stack([jnp.ones((CHUNK,), jnp.float32),
                       jnp.zeros((CHUNK,), jnp.float32)])
    zeros_rows = jnp.zeros((CHUNK, D), jnp.float32)

    cnt_flat = _cnt_call(dst_p, const)
    cnt_t = jnp.stack([cnt_flat[:N], cnt_flat[ACC_ROWS:ACC_ROWS + N]], axis=1)

    hp1 = _dense1(cnt_t, x.astype(jnp.float32), W1)
    s1 = _gs_call(hp1, src_p, dst_p, zeros_rows)
    hp2 = _dense2(cnt_t, s1[:N], s1[GS_ROWS:GS_ROWS + N], hp1,
                  b1.reshape(1, D), W2)
    s2 = _gs_call(hp2, src_p, dst_p, zeros_rows)
    h2, g = _dense3(cnt_t, s2[:N], s2[GS_ROWS:GS_ROWS + N], hp2,
                    b2.reshape(1, D))
    return h2, g


# speedup vs baseline: 1.0635x; 1.0635x over previous
import functools

import jax
import jax.numpy as jnp
from jax import lax
from jax.experimental import pallas as pl
from jax.experimental.pallas import tpu as pltpu
from jax.experimental.pallas import tpu_sc as plsc

N = 10000
D = 128
CHUNK = 128
NCHUNK = 80
NW = 32
CAP = NW * NCHUNK * CHUNK
ACC_ROWS = 10240
ZROWS = ACC_ROWS // 16
GROUP = 16
CH0 = 80
CH1 = 2 * NCHUNK - CH0
GS_ROWS = 10112
GZ = GS_ROWS // 16
RBLK = 2000

_mesh = plsc.VectorSubcoreMesh(core_axis_name="c", subcore_axis_name="s")


@functools.partial(
    pl.kernel,
    out_type=jax.ShapeDtypeStruct((2 * ACC_ROWS,), jnp.float32),
    mesh=_mesh,
    scratch_types=[
        pltpu.VMEM((NCHUNK, CHUNK), jnp.int32),
        pltpu.VMEM((2, CHUNK), jnp.float32),
        pltpu.VMEM_SHARED((ACC_ROWS,), jnp.float32),
    ],
)
def _cnt_call(dst_hbm, const_hbm, out_hbm, dst_v, const_v, acc):
    c = lax.axis_index("c")
    s = lax.axis_index("s")
    w = s * 2 + c
    pltpu.sync_copy(dst_hbm.at[pl.ds(w * NCHUNK, NCHUNK)], dst_v)
    pltpu.sync_copy(const_hbm, const_v)
    for j in range(ZROWS // CHUNK):
        pltpu.sync_copy(const_v.at[1], acc.at[pl.ds(s * ZROWS + j * CHUNK, CHUNK)])
    plsc.subcore_barrier()

    def body(i, _):
        pltpu.sync_copy(const_v.at[0], acc.at[dst_v.at[i]], add=True)
        return 0

    lax.fori_loop(0, NCHUNK, body, 0)
    plsc.subcore_barrier()
    pltpu.sync_copy(acc.at[pl.ds(s * ZROWS, ZROWS)],
                    out_hbm.at[pl.ds(c * ACC_ROWS + s * ZROWS, ZROWS)])


@functools.partial(
    pl.kernel,
    out_type=jax.ShapeDtypeStruct((2 * GS_ROWS, D), jnp.float32),
    mesh=_mesh,
    scratch_types=[
        pltpu.VMEM((2, GROUP, CHUNK), jnp.int32),
        pltpu.VMEM((2, GROUP, CHUNK), jnp.int32),
        pltpu.VMEM((CHUNK, D), jnp.float32),
        pltpu.VMEM((CHUNK, D), jnp.float32),
        pltpu.VMEM_SHARED((GS_ROWS, D), jnp.float32),
        pltpu.SemaphoreType.DMA,
        pltpu.SemaphoreType.DMA,
        pltpu.SemaphoreType.DMA,
    ],
)
def _gs_call(hp_hbm, src_hbm, dst_hbm, zeros_hbm, out_hbm,
             sidx, didx, r0, r1, acc, semi, sem0, sem1):
    c = lax.axis_index("c")
    s = lax.axis_index("s")

    pltpu.sync_copy(zeros_hbm, r0)
    for j in range(4):
        pltpu.sync_copy(r0, acc.at[pl.ds(s * GZ + j * CHUNK, CHUNK)])
    pltpu.sync_copy(r0.at[pl.ds(0, GZ - 4 * CHUNK)],
                    acc.at[pl.ds(s * GZ + 4 * CHUNK, GZ - 4 * CHUNK)])
    plsc.subcore_barrier()

    def pipeline(base, ngroup):
        if ngroup == 0:
            return
        def idx_copies(g):
            b = g % 2
            rows = pl.ds(base + g * GROUP, GROUP)
            return (pltpu.make_async_copy(src_hbm.at[rows], sidx.at[b], semi),
                    pltpu.make_async_copy(dst_hbm.at[rows], didx.at[b], semi))

        for cp in idx_copies(0):
            cp.start()
        for g in range(ngroup):
            for cp in idx_copies(g):
                cp.wait()
            if g + 1 < ngroup:
                for cp in idx_copies(g + 1):
                    cp.start()
            sv = sidx.at[g % 2]
            dv = didx.at[g % 2]
            pltpu.async_copy(hp_hbm.at[sv.at[0]], r0, sem0)
            pltpu.async_copy(hp_hbm.at[sv.at[1]], r1, sem1)

            def body(p, _, sv=sv, dv=dv):
                i = 2 * p
                pltpu.make_async_copy(hp_hbm.at[sv.at[i]], r0, sem0).wait()
                pltpu.sync_copy(r0, acc.at[dv.at[i]], add=True)
                pltpu.async_copy(hp_hbm.at[sv.at[i + 2]], r0, sem0)
                pltpu.make_async_copy(hp_hbm.at[sv.at[i + 1]], r1, sem1).wait()
                pltpu.sync_copy(r1, acc.at[dv.at[i + 1]], add=True)
                pltpu.async_copy(hp_hbm.at[sv.at[i + 3]], r1, sem1)
                return 0

            lax.fori_loop(0, GROUP // 2 - 1, body, 0)
            pltpu.make_async_copy(hp_hbm.at[sv.at[GROUP - 2]], r0, sem0).wait()
            pltpu.sync_copy(r0, acc.at[dv.at[GROUP - 2]], add=True)
            pltpu.make_async_copy(hp_hbm.at[sv.at[GROUP - 1]], r1, sem1).wait()
            pltpu.sync_copy(r1, acc.at[dv.at[GROUP - 1]], add=True)

    @pl.when(c == 0)
    def _():
        pipeline(s * CH0, CH0 // GROUP)

    @pl.when(c == 1)
    def _():
        pipeline(16 * CH0 + s * CH1, CH1 // GROUP)

    plsc.subcore_barrier()
    pltpu.sync_copy(acc.at[pl.ds(s * GZ, GZ)],
                    out_hbm.at[pl.ds(c * GS_ROWS + s * GZ, GZ)])


def _dis(cnt_ref):
    deg = cnt_ref[:, 0:1] + cnt_ref[:, 1:2] + 1.0
    return lax.rsqrt(deg)


def _dense1_body(cnt_ref, x_ref, w_ref, o_ref):
    xw = jnp.dot(x_ref[...], w_ref[...], preferred_element_type=jnp.float32,
                 precision=lax.Precision.HIGHEST)
    o_ref[...] = xw * _dis(cnt_ref)


def _dense2_body(cnt_ref, s0_ref, s1_ref, hp_ref, b_ref, w_ref, o_ref):
    dis = _dis(cnt_ref)
    h1 = jnp.maximum(dis * (s0_ref[...] + s1_ref[...] + hp_ref[...]) + b_ref[...],
                     0.0)
    o_ref[...] = dis * jnp.dot(h1, w_ref[...], preferred_element_type=jnp.float32,
                               precision=lax.Precision.HIGHEST)


def _dense3_body(cnt_ref, s0_ref, s1_ref, hp_ref, b_ref, h_ref, g_ref):
    dis = _dis(cnt_ref)
    h2 = dis * (s0_ref[...] + s1_ref[...] + hp_ref[...]) + b_ref[...]
    h_ref[...] = h2

    @pl.when(pl.program_id(0) == 0)
    def _():
        g_ref[...] = jnp.zeros_like(g_ref)

    g_ref[...] += jnp.sum(h2, axis=0, keepdims=True) * (1.0 / N)


_row_spec = pl.BlockSpec((RBLK, D), lambda i: (i, 0))
_cnt_spec = pl.BlockSpec((RBLK, 2), lambda i: (i, 0))
_full_spec = pl.BlockSpec((D, D), lambda i: (0, 0))
_b_spec = pl.BlockSpec((1, D), lambda i: (0, 0))

_dense1 = pl.pallas_call(
    _dense1_body, grid=(N // RBLK,),
    in_specs=[_cnt_spec, _row_spec, _full_spec],
    out_specs=_row_spec,
    out_shape=jax.ShapeDtypeStruct((N, D), jnp.float32))

_dense2 = pl.pallas_call(
    _dense2_body, grid=(N // RBLK,),
    in_specs=[_cnt_spec, _row_spec, _row_spec, _row_spec, _b_spec, _full_spec],
    out_specs=_row_spec,
    out_shape=jax.ShapeDtypeStruct((N, D), jnp.float32))

_dense3 = pl.pallas_call(
    _dense3_body, grid=(N // RBLK,),
    in_specs=[_cnt_spec, _row_spec, _row_spec, _row_spec, _b_spec],
    out_specs=[_row_spec, pl.BlockSpec((1, D), lambda i: (0, 0))],
    out_shape=[jax.ShapeDtypeStruct((N, D), jnp.float32),
               jax.ShapeDtypeStruct((1, D), jnp.float32)])


def kernel(x, edge_index, W1, b1, W2, b2):
    src = edge_index[0].astype(jnp.int32)
    dst = edge_index[1].astype(jnp.int32)
    e = src.shape[0]
    pad = CAP - e
    src_p = jnp.concatenate([src, jnp.zeros((pad,), jnp.int32)]
                            ).reshape(NW * NCHUNK, CHUNK)
    pad_dst = N + jnp.arange(pad, dtype=jnp.int32) % (GS_ROWS - N)
    dst_p = jnp.concatenate([dst, pad_dst]).reshape(NW * NCHUNK, CHUNK)
    const = jnp.stack([jnp.ones((CHUNK,), jnp.float32),
                       jnp.zeros((CHUNK,), jnp.float32)])
    zeros_rows = jnp.zeros((CHUNK, D), jnp.float32)

    cnt_flat = _cnt_call(dst_p, const)
    cnt_t = jnp.stack([cnt_flat[:N], cnt_flat[ACC_ROWS:ACC_ROWS + N]], axis=1)

    hp1 = _dense1(cnt_t, x.astype(jnp.float32), W1)
    s1 = _gs_call(hp1, src_p, dst_p, zeros_rows)
    hp2 = _dense2(cnt_t, s1[:N], s1[GS_ROWS:GS_ROWS + N], hp1,
                  b1.reshape(1, D), W2)
    s2 = _gs_call(hp2, src_p, dst_p, zeros_rows)
    h2, g = _dense3(cnt_t, s2[:N], s2[GS_ROWS:GS_ROWS + N], hp2,
                    b2.reshape(1, D))
    return h2, g


# speedup vs baseline: 1.1727x; 1.1028x over previous
import functools

import jax
import jax.numpy as jnp
from jax import lax
from jax.experimental import pallas as pl
from jax.experimental.pallas import tpu as pltpu
from jax.experimental.pallas import tpu_sc as plsc

N = 10000
D = 128
CHUNK = 128
NCHUNK = 80
NW = 32
CAP = NW * NCHUNK * CHUNK
ACC_ROWS = 10240
ZROWS = ACC_ROWS // 16
GROUP = 16
CH0 = 128
CH1 = 2 * NCHUNK - CH0
GS_ROWS = 10112
GZ = GS_ROWS // 16
RBLK = 2000

_mesh = plsc.VectorSubcoreMesh(core_axis_name="c", subcore_axis_name="s")


@functools.partial(
    pl.kernel,
    out_type=jax.ShapeDtypeStruct((2 * ACC_ROWS,), jnp.float32),
    mesh=_mesh,
    scratch_types=[
        pltpu.VMEM((NCHUNK, CHUNK), jnp.int32),
        pltpu.VMEM((2, CHUNK), jnp.float32),
        pltpu.VMEM_SHARED((ACC_ROWS,), jnp.float32),
    ],
)
def _cnt_call(dst_hbm, const_hbm, out_hbm, dst_v, const_v, acc):
    c = lax.axis_index("c")
    s = lax.axis_index("s")
    w = s * 2 + c
    pltpu.sync_copy(dst_hbm.at[pl.ds(w * NCHUNK, NCHUNK)], dst_v)
    pltpu.sync_copy(const_hbm, const_v)
    for j in range(ZROWS // CHUNK):
        pltpu.sync_copy(const_v.at[1], acc.at[pl.ds(s * ZROWS + j * CHUNK, CHUNK)])
    plsc.subcore_barrier()

    def body(i, _):
        pltpu.sync_copy(const_v.at[0], acc.at[dst_v.at[i]], add=True)
        return 0

    lax.fori_loop(0, NCHUNK, body, 0)
    plsc.subcore_barrier()
    pltpu.sync_copy(acc.at[pl.ds(s * ZROWS, ZROWS)],
                    out_hbm.at[pl.ds(c * ACC_ROWS + s * ZROWS, ZROWS)])


@functools.partial(
    pl.kernel,
    out_type=jax.ShapeDtypeStruct((2 * GS_ROWS, D), jnp.float32),
    mesh=_mesh,
    scratch_types=[
        pltpu.VMEM((2, GROUP, CHUNK), jnp.int32),
        pltpu.VMEM((2, GROUP, CHUNK), jnp.int32),
        pltpu.VMEM((CHUNK, D), jnp.float32),
        pltpu.VMEM((CHUNK, D), jnp.float32),
        pltpu.VMEM_SHARED((GS_ROWS, D), jnp.float32),
        pltpu.SemaphoreType.DMA,
        pltpu.SemaphoreType.DMA,
        pltpu.SemaphoreType.DMA,
    ],
)
def _gs_call(hp_hbm, src_hbm, dst_hbm, zeros_hbm, out_hbm,
             sidx, didx, r0, r1, acc, semi, sem0, sem1):
    c = lax.axis_index("c")
    s = lax.axis_index("s")

    pltpu.sync_copy(zeros_hbm, r0)
    for j in range(4):
        pltpu.sync_copy(r0, acc.at[pl.ds(s * GZ + j * CHUNK, CHUNK)])
    pltpu.sync_copy(r0.at[pl.ds(0, GZ - 4 * CHUNK)],
                    acc.at[pl.ds(s * GZ + 4 * CHUNK, GZ - 4 * CHUNK)])
    plsc.subcore_barrier()

    def pipeline(base, ngroup):
        if ngroup == 0:
            return
        def idx_copies(g):
            b = g % 2
            rows = pl.ds(base + g * GROUP, GROUP)
            return (pltpu.make_async_copy(src_hbm.at[rows], sidx.at[b], semi),
                    pltpu.make_async_copy(dst_hbm.at[rows], didx.at[b], semi))

        for cp in idx_copies(0):
            cp.start()
        for g in range(ngroup):
            for cp in idx_copies(g):
                cp.wait()
            if g + 1 < ngroup:
                for cp in idx_copies(g + 1):
                    cp.start()
            sv = sidx.at[g % 2]
            dv = didx.at[g % 2]
            pltpu.async_copy(hp_hbm.at[sv.at[0]], r0, sem0)
            pltpu.async_copy(hp_hbm.at[sv.at[1]], r1, sem1)

            def body(p, _, sv=sv, dv=dv):
                i = 2 * p
                pltpu.make_async_copy(hp_hbm.at[sv.at[i]], r0, sem0).wait()
                pltpu.sync_copy(r0, acc.at[dv.at[i]], add=True)
                pltpu.async_copy(hp_hbm.at[sv.at[i + 2]], r0, sem0)
                pltpu.make_async_copy(hp_hbm.at[sv.at[i + 1]], r1, sem1).wait()
                pltpu.sync_copy(r1, acc.at[dv.at[i + 1]], add=True)
                pltpu.async_copy(hp_hbm.at[sv.at[i + 3]], r1, sem1)
                return 0

            lax.fori_loop(0, GROUP // 2 - 1, body, 0)
            pltpu.make_async_copy(hp_hbm.at[sv.at[GROUP - 2]], r0, sem0).wait()
            pltpu.sync_copy(r0, acc.at[dv.at[GROUP - 2]], add=True)
            pltpu.make_async_copy(hp_hbm.at[sv.at[GROUP - 1]], r1, sem1).wait()
            pltpu.sync_copy(r1, acc.at[dv.at[GROUP - 1]], add=True)

    @pl.when(c == 0)
    def _():
        pipeline(s * CH0, CH0 // GROUP)

    @pl.when(c == 1)
    def _():
        pipeline(16 * CH0 + s * CH1, CH1 // GROUP)

    plsc.subcore_barrier()
    pltpu.sync_copy(acc.at[pl.ds(s * GZ, GZ)],
                    out_hbm.at[pl.ds(c * GS_ROWS + s * GZ, GZ)])


def _dis(cnt_ref):
    deg = cnt_ref[:, 0:1] + cnt_ref[:, 1:2] + 1.0
    return lax.rsqrt(deg)


def _dense1_body(cnt_ref, x_ref, w_ref, o_ref):
    xw = jnp.dot(x_ref[...], w_ref[...], preferred_element_type=jnp.float32,
                 precision=lax.Precision.HIGHEST)
    o_ref[...] = xw * _dis(cnt_ref)


def _dense2_body(cnt_ref, s0_ref, s1_ref, hp_ref, b_ref, w_ref, o_ref):
    dis = _dis(cnt_ref)
    h1 = jnp.maximum(dis * (s0_ref[...] + s1_ref[...] + hp_ref[...]) + b_ref[...],
                     0.0)
    o_ref[...] = dis * jnp.dot(h1, w_ref[...], preferred_element_type=jnp.float32,
                               precision=lax.Precision.HIGHEST)


def _dense3_body(cnt_ref, s0_ref, s1_ref, hp_ref, b_ref, h_ref, g_ref):
    dis = _dis(cnt_ref)
    h2 = dis * (s0_ref[...] + s1_ref[...] + hp_ref[...]) + b_ref[...]
    h_ref[...] = h2

    @pl.when(pl.program_id(0) == 0)
    def _():
        g_ref[...] = jnp.zeros_like(g_ref)

    g_ref[...] += jnp.sum(h2, axis=0, keepdims=True) * (1.0 / N)


_row_spec = pl.BlockSpec((RBLK, D), lambda i: (i, 0))
_cnt_spec = pl.BlockSpec((RBLK, 2), lambda i: (i, 0))
_full_spec = pl.BlockSpec((D, D), lambda i: (0, 0))
_b_spec = pl.BlockSpec((1, D), lambda i: (0, 0))

_dense1 = pl.pallas_call(
    _dense1_body, grid=(N // RBLK,),
    in_specs=[_cnt_spec, _row_spec, _full_spec],
    out_specs=_row_spec,
    out_shape=jax.ShapeDtypeStruct((N, D), jnp.float32))

_dense2 = pl.pallas_call(
    _dense2_body, grid=(N // RBLK,),
    in_specs=[_cnt_spec, _row_spec, _row_spec, _row_spec, _b_spec, _full_spec],
    out_specs=_row_spec,
    out_shape=jax.ShapeDtypeStruct((N, D), jnp.float32))

_dense3 = pl.pallas_call(
    _dense3_body, grid=(N // RBLK,),
    in_specs=[_cnt_spec, _row_spec, _row_spec, _row_spec, _b_spec],
    out_specs=[_row_spec, pl.BlockSpec((1, D), lambda i: (0, 0))],
    out_shape=[jax.ShapeDtypeStruct((N, D), jnp.float32),
               jax.ShapeDtypeStruct((1, D), jnp.float32)])


def kernel(x, edge_index, W1, b1, W2, b2):
    src = edge_index[0].astype(jnp.int32)
    dst = edge_index[1].astype(jnp.int32)
    e = src.shape[0]
    pad = CAP - e
    src_p = jnp.concatenate([src, jnp.zeros((pad,), jnp.int32)]
                            ).reshape(NW * NCHUNK, CHUNK)
    pad_dst = N + jnp.arange(pad, dtype=jnp.int32) % (GS_ROWS - N)
    dst_p = jnp.concatenate([dst, pad_dst]).reshape(NW * NCHUNK, CHUNK)
    const = jnp.stack([jnp.ones((CHUNK,), jnp.float32),
                       jnp.zeros((CHUNK,), jnp.float32)])
    zeros_rows = jnp.zeros((CHUNK, D), jnp.float32)

    cnt_flat = _cnt_call(dst_p, const)
    cnt_t = jnp.stack([cnt_flat[:N], cnt_flat[ACC_ROWS:ACC_ROWS + N]], axis=1)

    hp1 = _dense1(cnt_t, x.astype(jnp.float32), W1)
    s1 = _gs_call(hp1, src_p, dst_p, zeros_rows)
    hp2 = _dense2(cnt_t, s1[:N], s1[GS_ROWS:GS_ROWS + N], hp1,
                  b1.reshape(1, D), W2)
    s2 = _gs_call(hp2, src_p, dst_p, zeros_rows)
    h2, g = _dense3(cnt_t, s2[:N], s2[GS_ROWS:GS_ROWS + N], hp2,
                    b2.reshape(1, D))
    return h2, g


# speedup vs baseline: 3.5210x; 3.0024x over previous
import functools

import jax
import jax.numpy as jnp
from jax import lax
from jax.experimental import pallas as pl
from jax.experimental.pallas import tpu as pltpu
from jax.experimental.pallas import tpu_sc as plsc

N = 10000
D = 128
CHUNK = 128
NCHUNK = 80
NW = 32
CAP = NW * NCHUNK * CHUNK
ACC_ROWS = 10240
ZROWS = ACC_ROWS // 16
GROUP = 16
CH0 = 80
CH1 = 2 * NCHUNK - CH0
GS_ROWS = 10112
GZ = GS_ROWS // 16
RBLK = 2000

_mesh = plsc.VectorSubcoreMesh(core_axis_name="c", subcore_axis_name="s")


@functools.partial(
    pl.kernel,
    out_type=jax.ShapeDtypeStruct((2 * ACC_ROWS,), jnp.float32),
    mesh=_mesh,
    scratch_types=[
        pltpu.VMEM((NCHUNK, CHUNK), jnp.int32),
        pltpu.VMEM((2, CHUNK), jnp.float32),
        pltpu.VMEM_SHARED((ACC_ROWS,), jnp.float32),
    ],
)
def _cnt_call(dst_hbm, const_hbm, out_hbm, dst_v, const_v, acc):
    c = lax.axis_index("c")
    s = lax.axis_index("s")
    w = s * 2 + c
    pltpu.sync_copy(dst_hbm.at[pl.ds(w * NCHUNK, NCHUNK)], dst_v)
    pltpu.sync_copy(const_hbm, const_v)
    for j in range(ZROWS // CHUNK):
        pltpu.sync_copy(const_v.at[1], acc.at[pl.ds(s * ZROWS + j * CHUNK, CHUNK)])
    plsc.subcore_barrier()

    def body(i, _):
        pltpu.sync_copy(const_v.at[0], acc.at[dst_v.at[i]], add=True)
        return 0

    lax.fori_loop(0, NCHUNK, body, 0)
    plsc.subcore_barrier()
    pltpu.sync_copy(acc.at[pl.ds(s * ZROWS, ZROWS)],
                    out_hbm.at[pl.ds(c * ACC_ROWS + s * ZROWS, ZROWS)])


@functools.partial(
    pl.kernel,
    out_type=jax.ShapeDtypeStruct((2 * GS_ROWS, D), jnp.float32),
    mesh=_mesh,
    scratch_types=[
        pltpu.VMEM((2, GROUP, CHUNK), jnp.int32),
        pltpu.VMEM((2, GROUP, CHUNK), jnp.int32),
        pltpu.VMEM((CHUNK, D), jnp.float32),
        pltpu.VMEM((CHUNK, D), jnp.float32),
        pltpu.VMEM_SHARED((GS_ROWS, D), jnp.float32),
        pltpu.SemaphoreType.DMA,
        pltpu.SemaphoreType.DMA,
        pltpu.SemaphoreType.DMA,
    ],
)
def _gs_call(hp_hbm, src_hbm, dst_hbm, zeros_hbm, out_hbm,
             sidx, didx, r0, r1, acc, semi, sem0, sem1):
    c = lax.axis_index("c")
    s = lax.axis_index("s")

    pltpu.sync_copy(zeros_hbm, r0)
    for j in range(4):
        pltpu.sync_copy(r0, acc.at[pl.ds(s * GZ + j * CHUNK, CHUNK)])
    pltpu.sync_copy(r0.at[pl.ds(0, GZ - 4 * CHUNK)],
                    acc.at[pl.ds(s * GZ + 4 * CHUNK, GZ - 4 * CHUNK)])
    plsc.subcore_barrier()

    def pipeline(base, ngroup, scatter=True):
        if ngroup == 0:
            return
        def idx_copies(g):
            b = g % 2
            rows = pl.ds(base + g * GROUP, GROUP)
            return (pltpu.make_async_copy(src_hbm.at[rows], sidx.at[b], semi),
                    pltpu.make_async_copy(dst_hbm.at[rows], didx.at[b], semi))

        for cp in idx_copies(0):
            cp.start()
        for g in range(ngroup):
            for cp in idx_copies(g):
                cp.wait()
            if g + 1 < ngroup:
                for cp in idx_copies(g + 1):
                    cp.start()
            sv = sidx.at[g % 2]
            dv = didx.at[g % 2]
            pltpu.async_copy(hp_hbm.at[sv.at[0]], r0, sem0)
            pltpu.async_copy(hp_hbm.at[sv.at[1]], r1, sem1)

            def body(p, _, sv=sv, dv=dv):
                i = 2 * p
                pltpu.make_async_copy(hp_hbm.at[sv.at[i]], r0, sem0).wait()
                if scatter:
                    pltpu.sync_copy(r0, acc.at[dv.at[i]], add=True)
                pltpu.async_copy(hp_hbm.at[sv.at[i + 2]], r0, sem0)
                pltpu.make_async_copy(hp_hbm.at[sv.at[i + 1]], r1, sem1).wait()
                if scatter:
                    pltpu.sync_copy(r1, acc.at[dv.at[i + 1]], add=True)
                pltpu.async_copy(hp_hbm.at[sv.at[i + 3]], r1, sem1)
                return 0

            lax.fori_loop(0, GROUP // 2 - 1, body, 0)
            pltpu.make_async_copy(hp_hbm.at[sv.at[GROUP - 2]], r0, sem0).wait()
            if scatter:
                pltpu.sync_copy(r0, acc.at[dv.at[GROUP - 2]], add=True)
            pltpu.make_async_copy(hp_hbm.at[sv.at[GROUP - 1]], r1, sem1).wait()
            if scatter:
                pltpu.sync_copy(r1, acc.at[dv.at[GROUP - 1]], add=True)

    @pl.when(c == 0)
    def _():
        pipeline(s * CH0, CH0 // GROUP)

    @pl.when(c == 1)
    def _():
        pipeline(16 * CH0 + s * CH1, CH1 // GROUP)

    plsc.subcore_barrier()
    pltpu.sync_copy(acc.at[pl.ds(s * GZ, GZ)],
                    out_hbm.at[pl.ds(c * GS_ROWS + s * GZ, GZ)])


def _dis(cnt_ref):
    deg = cnt_ref[:, 0:1] + cnt_ref[:, 1:2] + 1.0
    return lax.rsqrt(deg)


def _dense1_body(cnt_ref, x_ref, w_ref, o_ref):
    xw = jnp.dot(x_ref[...], w_ref[...], preferred_element_type=jnp.float32,
                 precision=lax.Precision.HIGHEST)
    o_ref[...] = xw * _dis(cnt_ref)


def _dense2_body(cnt_ref, s0_ref, s1_ref, hp_ref, b_ref, w_ref, o_ref):
    dis = _dis(cnt_ref)
    h1 = jnp.maximum(dis * (s0_ref[...] + s1_ref[...] + hp_ref[...]) + b_ref[...],
                     0.0)
    o_ref[...] = dis * jnp.dot(h1, w_ref[...], preferred_element_type=jnp.float32,
                               precision=lax.Precision.HIGHEST)


def _dense3_body(cnt_ref, s0_ref, s1_ref, hp_ref, b_ref, h_ref, g_ref):
    dis = _dis(cnt_ref)
    h2 = dis * (s0_ref[...] + s1_ref[...] + hp_ref[...]) + b_ref[...]
    h_ref[...] = h2

    @pl.when(pl.program_id(0) == 0)
    def _():
        g_ref[...] = jnp.zeros_like(g_ref)

    g_ref[...] += jnp.sum(h2, axis=0, keepdims=True) * (1.0 / N)


_row_spec = pl.BlockSpec((RBLK, D), lambda i: (i, 0))
_cnt_spec = pl.BlockSpec((RBLK, 2), lambda i: (i, 0))
_full_spec = pl.BlockSpec((D, D), lambda i: (0, 0))
_b_spec = pl.BlockSpec((1, D), lambda i: (0, 0))

_dense1 = pl.pallas_call(
    _dense1_body, grid=(N // RBLK,),
    in_specs=[_cnt_spec, _row_spec, _full_spec],
    out_specs=_row_spec,
    out_shape=jax.ShapeDtypeStruct((N, D), jnp.float32))

_dense2 = pl.pallas_call(
    _dense2_body, grid=(N // RBLK,),
    in_specs=[_cnt_spec, _row_spec, _row_spec, _row_spec, _b_spec, _full_spec],
    out_specs=_row_spec,
    out_shape=jax.ShapeDtypeStruct((N, D), jnp.float32))

_dense3 = pl.pallas_call(
    _dense3_body, grid=(N // RBLK,),
    in_specs=[_cnt_spec, _row_spec, _row_spec, _row_spec, _b_spec],
    out_specs=[_row_spec, pl.BlockSpec((1, D), lambda i: (0, 0))],
    out_shape=[jax.ShapeDtypeStruct((N, D), jnp.float32),
               jax.ShapeDtypeStruct((1, D), jnp.float32)])


def kernel(x, edge_index, W1, b1, W2, b2):
    src = edge_index[0].astype(jnp.int32)
    dst = edge_index[1].astype(jnp.int32)
    e = src.shape[0]
    pad = CAP - e
    pad_idx = jnp.arange(pad, dtype=jnp.int32)
    src_p = jnp.concatenate([src, pad_idx % N]).reshape(NW * NCHUNK, CHUNK)
    pad_dst = N + pad_idx % (GS_ROWS - N)
    dst_p = jnp.concatenate([dst, pad_dst]).reshape(NW * NCHUNK, CHUNK)
    const = jnp.stack([jnp.ones((CHUNK,), jnp.float32),
                       jnp.zeros((CHUNK,), jnp.float32)])
    zeros_rows = jnp.zeros((CHUNK, D), jnp.float32)

    cnt_flat = _cnt_call(dst_p, const)
    cnt_t = jnp.stack([cnt_flat[:N], cnt_flat[ACC_ROWS:ACC_ROWS + N]], axis=1)

    hp1 = _dense1(cnt_t, x.astype(jnp.float32), W1)
    s1 = _gs_call(hp1, src_p, dst_p, zeros_rows)
    hp2 = _dense2(cnt_t, s1[:N], s1[GS_ROWS:GS_ROWS + N], hp1,
                  b1.reshape(1, D), W2)
    s2 = _gs_call(hp2, src_p, dst_p, zeros_rows)
    h2, g = _dense3(cnt_t, s2[:N], s2[GS_ROWS:GS_ROWS + N], hp2,
                    b2.reshape(1, D))
    return h2, g


# speedup vs baseline: 3.8205x; 1.0851x over previous
import functools

import jax
import jax.numpy as jnp
from jax import lax
from jax.experimental import pallas as pl
from jax.experimental.pallas import tpu as pltpu
from jax.experimental.pallas import tpu_sc as plsc

N = 10000
D = 128
NW = 32
CAP = 327680
CHUNK = 64
NCHUNK = 160
NB = 4
GROUP = 32
CH0 = 160
CH1 = 2 * NCHUNK - CH0
CNT_CHUNK = 128
CNT_NCHUNK = CAP // (NW * CNT_CHUNK)
ACC_ROWS = 10240
ZROWS = ACC_ROWS // 16
GS_ROWS = 10112
GZ = GS_ROWS // 16
RBLK = 2000

_mesh = plsc.VectorSubcoreMesh(core_axis_name="c", subcore_axis_name="s")


@functools.partial(
    pl.kernel,
    out_type=jax.ShapeDtypeStruct((2 * ACC_ROWS,), jnp.float32),
    mesh=_mesh,
    scratch_types=[
        pltpu.VMEM((CNT_NCHUNK, CNT_CHUNK), jnp.int32),
        pltpu.VMEM((2, CNT_CHUNK), jnp.float32),
        pltpu.VMEM_SHARED((ACC_ROWS,), jnp.float32),
    ],
)
def _cnt_call(dst_hbm, const_hbm, out_hbm, dst_v, const_v, acc):
    c = lax.axis_index("c")
    s = lax.axis_index("s")
    w = s * 2 + c
    pltpu.sync_copy(dst_hbm.at[pl.ds(w * CNT_NCHUNK, CNT_NCHUNK)], dst_v)
    pltpu.sync_copy(const_hbm, const_v)
    for j in range(ZROWS // CNT_CHUNK):
        pltpu.sync_copy(const_v.at[1],
                        acc.at[pl.ds(s * ZROWS + j * CNT_CHUNK, CNT_CHUNK)])
    plsc.subcore_barrier()

    def body(i, _):
        pltpu.sync_copy(const_v.at[0], acc.at[dst_v.at[i]], add=True)
        return 0

    lax.fori_loop(0, CNT_NCHUNK, body, 0)
    plsc.subcore_barrier()
    pltpu.sync_copy(acc.at[pl.ds(s * ZROWS, ZROWS)],
                    out_hbm.at[pl.ds(c * ACC_ROWS + s * ZROWS, ZROWS)])


@functools.partial(
    pl.kernel,
    out_type=jax.ShapeDtypeStruct((2 * GS_ROWS, D), jnp.float32),
    mesh=_mesh,
    scratch_types=[
        pltpu.VMEM((2, GROUP, CHUNK), jnp.int32),
        pltpu.VMEM((2, GROUP, CHUNK), jnp.int32),
        [pltpu.VMEM((CHUNK, D), jnp.float32)] * NB,
        pltpu.VMEM_SHARED((GS_ROWS, D), jnp.float32),
        pltpu.SemaphoreType.DMA,
        [pltpu.SemaphoreType.DMA] * NB,
    ],
)
def _gs_call(hp_hbm, src_hbm, dst_hbm, zeros_hbm, out_hbm,
             sidx, didx, bufs, acc, semi, sems):
    c = lax.axis_index("c")
    s = lax.axis_index("s")

    pltpu.sync_copy(zeros_hbm, bufs[0])
    for j in range(GZ // CHUNK):
        pltpu.sync_copy(bufs[0], acc.at[pl.ds(s * GZ + j * CHUNK, CHUNK)])
    tail = GZ % CHUNK
    pltpu.sync_copy(bufs[0].at[pl.ds(0, tail)],
                    acc.at[pl.ds(s * GZ + GZ - tail, tail)])
    plsc.subcore_barrier()

    def pipeline(base, ngroup):
        if ngroup == 0:
            return
        def idx_copies(g):
            b = g % 2
            rows = pl.ds(base + g * GROUP, GROUP)
            return (pltpu.make_async_copy(src_hbm.at[rows], sidx.at[b], semi),
                    pltpu.make_async_copy(dst_hbm.at[rows], didx.at[b], semi))

        for cp in idx_copies(0):
            cp.start()
        for g in range(ngroup):
            for cp in idx_copies(g):
                cp.wait()
            if g + 1 < ngroup:
                for cp in idx_copies(g + 1):
                    cp.start()
            sv = sidx.at[g % 2]
            dv = didx.at[g % 2]
            for b in range(NB):
                pltpu.async_copy(hp_hbm.at[sv.at[b]], bufs[b], sems[b])

            def body(p, _, sv=sv, dv=dv):
                i = NB * p
                for b in range(NB):
                    pltpu.make_async_copy(
                        hp_hbm.at[sv.at[i + b]], bufs[b], sems[b]).wait()
                    pltpu.sync_copy(bufs[b], acc.at[dv.at[i + b]], add=True)
                    pltpu.async_copy(
                        hp_hbm.at[sv.at[i + NB + b]], bufs[b], sems[b])
                return 0

            lax.fori_loop(0, GROUP // NB - 1, body, 0)
            for b in range(NB):
                i = GROUP - NB + b
                pltpu.make_async_copy(
                    hp_hbm.at[sv.at[i]], bufs[b], sems[b]).wait()
                pltpu.sync_copy(bufs[b], acc.at[dv.at[i]], add=True)

    @pl.when(c == 0)
    def _():
        pipeline(s * CH0, CH0 // GROUP)

    @pl.when(c == 1)
    def _():
        pipeline(16 * CH0 + s * CH1, CH1 // GROUP)

    plsc.subcore_barrier()
    pltpu.sync_copy(acc.at[pl.ds(s * GZ, GZ)],
                    out_hbm.at[pl.ds(c * GS_ROWS + s * GZ, GZ)])


def _dis(cnt_ref):
    deg = cnt_ref[:, 0:1] + cnt_ref[:, 1:2] + 1.0
    return lax.rsqrt(deg)


def _dense1_body(cnt_ref, x_ref, w_ref, o_ref):
    xw = jnp.dot(x_ref[...], w_ref[...], preferred_element_type=jnp.float32,
                 precision=lax.Precision.HIGHEST)
    o_ref[...] = xw * _dis(cnt_ref)


def _dense2_body(cnt_ref, s0_ref, s1_ref, hp_ref, b_ref, w_ref, o_ref):
    dis = _dis(cnt_ref)
    h1 = jnp.maximum(dis * (s0_ref[...] + s1_ref[...] + hp_ref[...]) + b_ref[...],
                     0.0)
    o_ref[...] = dis * jnp.dot(h1, w_ref[...], preferred_element_type=jnp.float32,
                               precision=lax.Precision.HIGHEST)


def _dense3_body(cnt_ref, s0_ref, s1_ref, hp_ref, b_ref, h_ref, g_ref):
    dis = _dis(cnt_ref)
    h2 = dis * (s0_ref[...] + s1_ref[...] + hp_ref[...]) + b_ref[...]
    h_ref[...] = h2

    @pl.when(pl.program_id(0) == 0)
    def _():
        g_ref[...] = jnp.zeros_like(g_ref)

    g_ref[...] += jnp.sum(h2, axis=0, keepdims=True) * (1.0 / N)


_row_spec = pl.BlockSpec((RBLK, D), lambda i: (i, 0))
_cnt_spec = pl.BlockSpec((RBLK, 2), lambda i: (i, 0))
_full_spec = pl.BlockSpec((D, D), lambda i: (0, 0))
_b_spec = pl.BlockSpec((1, D), lambda i: (0, 0))

_dense1 = pl.pallas_call(
    _dense1_body, grid=(N // RBLK,),
    in_specs=[_cnt_spec, _row_spec, _full_spec],
    out_specs=_row_spec,
    out_shape=jax.ShapeDtypeStruct((N, D), jnp.float32))

_dense2 = pl.pallas_call(
    _dense2_body, grid=(N // RBLK,),
    in_specs=[_cnt_spec, _row_spec, _row_spec, _row_spec, _b_spec, _full_spec],
    out_specs=_row_spec,
    out_shape=jax.ShapeDtypeStruct((N, D), jnp.float32))

_dense3 = pl.pallas_call(
    _dense3_body, grid=(N // RBLK,),
    in_specs=[_cnt_spec, _row_spec, _row_spec, _row_spec, _b_spec],
    out_specs=[_row_spec, pl.BlockSpec((1, D), lambda i: (0, 0))],
    out_shape=[jax.ShapeDtypeStruct((N, D), jnp.float32),
               jax.ShapeDtypeStruct((1, D), jnp.float32)])


def kernel(x, edge_index, W1, b1, W2, b2):
    src = edge_index[0].astype(jnp.int32)
    dst = edge_index[1].astype(jnp.int32)
    e = src.shape[0]
    pad = CAP - e
    pad_idx = jnp.arange(pad, dtype=jnp.int32)
    src_p = jnp.concatenate([src, pad_idx % N]).reshape(NW * NCHUNK, CHUNK)
    dst_full = jnp.concatenate([dst, N + pad_idx % (GS_ROWS - N)])
    dst_p = dst_full.reshape(NW * NCHUNK, CHUNK)
    dst_p_cnt = dst_full.reshape(NW * CNT_NCHUNK, CNT_CHUNK)
    const = jnp.stack([jnp.ones((CNT_CHUNK,), jnp.float32),
                       jnp.zeros((CNT_CHUNK,), jnp.float32)])
    zeros_rows = jnp.zeros((CHUNK, D), jnp.float32)

    cnt_flat = _cnt_call(dst_p_cnt, const)
    cnt_t = jnp.stack([cnt_flat[:N], cnt_flat[ACC_ROWS:ACC_ROWS + N]], axis=1)

    hp1 = _dense1(cnt_t, x.astype(jnp.float32), W1)
    s1 = _gs_call(hp1, src_p, dst_p, zeros_rows)
    hp2 = _dense2(cnt_t, s1[:N], s1[GS_ROWS:GS_ROWS + N], hp1,
                  b1.reshape(1, D), W2)
    s2 = _gs_call(hp2, src_p, dst_p, zeros_rows)
    h2, g = _dense3(cnt_t, s2[:N], s2[GS_ROWS:GS_ROWS + N], hp2,
                    b2.reshape(1, D))
    return h2, g


# speedup vs baseline: 3.8700x; 1.0130x over previous
import functools

import jax
import jax.numpy as jnp
from jax import lax
from jax.experimental import pallas as pl
from jax.experimental.pallas import tpu as pltpu
from jax.experimental.pallas import tpu_sc as plsc

N = 10000
D = 128
NW = 32
CAP = 327680
CHUNK = 64
NCHUNK = 160
NB = 4
GROUP = 32
CH0 = 160
CH1 = 2 * NCHUNK - CH0
CNT_CHUNK = 128
CNT_NCHUNK = CAP // (NW * CNT_CHUNK)
ACC_ROWS = 10240
ZROWS = ACC_ROWS // 16
GS_ROWS = 10112
GZ = GS_ROWS // 16
RBLK = 2000

_mesh = plsc.VectorSubcoreMesh(core_axis_name="c", subcore_axis_name="s")


@functools.partial(
    pl.kernel,
    out_type=jax.ShapeDtypeStruct((2 * ACC_ROWS, 8), jnp.float32),
    mesh=_mesh,
    scratch_types=[
        pltpu.VMEM((CNT_NCHUNK, CNT_CHUNK), jnp.int32),
        pltpu.VMEM((2, CNT_CHUNK, 8), jnp.float32),
        pltpu.VMEM_SHARED((ACC_ROWS, 8), jnp.float32),
    ],
)
def _cnt_call(dst_hbm, const_hbm, out_hbm, dst_v, const_v, acc):
    c = lax.axis_index("c")
    s = lax.axis_index("s")
    w = s * 2 + c
    pltpu.sync_copy(dst_hbm.at[pl.ds(w * CNT_NCHUNK, CNT_NCHUNK)], dst_v)
    pltpu.sync_copy(const_hbm, const_v)
    for j in range(ZROWS // CNT_CHUNK):
        pltpu.sync_copy(const_v.at[1],
                        acc.at[pl.ds(s * ZROWS + j * CNT_CHUNK, CNT_CHUNK)])
    plsc.subcore_barrier()

    def body(i, _):
        pltpu.sync_copy(const_v.at[0], acc.at[dst_v.at[i]], add=True)
        return 0

    lax.fori_loop(0, CNT_NCHUNK, body, 0)
    plsc.subcore_barrier()
    pltpu.sync_copy(acc.at[pl.ds(s * ZROWS, ZROWS)],
                    out_hbm.at[pl.ds(c * ACC_ROWS + s * ZROWS, ZROWS)])


@functools.partial(
    pl.kernel,
    out_type=jax.ShapeDtypeStruct((2 * GS_ROWS, D), jnp.float32),
    mesh=_mesh,
    scratch_types=[
        pltpu.VMEM((2, GROUP, CHUNK), jnp.int32),
        pltpu.VMEM((2, GROUP, CHUNK), jnp.int32),
        [pltpu.VMEM((CHUNK, D), jnp.float32)] * NB,
        pltpu.VMEM_SHARED((GS_ROWS, D), jnp.float32),
        pltpu.SemaphoreType.DMA,
        [pltpu.SemaphoreType.DMA] * NB,
    ],
)
def _gs_call(hp_hbm, src_hbm, dst_hbm, zeros_hbm, out_hbm,
             sidx, didx, bufs, acc, semi, sems):
    c = lax.axis_index("c")
    s = lax.axis_index("s")

    pltpu.sync_copy(zeros_hbm, bufs[0])
    for j in range(GZ // CHUNK):
        pltpu.sync_copy(bufs[0], acc.at[pl.ds(s * GZ + j * CHUNK, CHUNK)])
    tail = GZ % CHUNK
    pltpu.sync_copy(bufs[0].at[pl.ds(0, tail)],
                    acc.at[pl.ds(s * GZ + GZ - tail, tail)])
    plsc.subcore_barrier()

    def pipeline(base, ngroup):
        if ngroup == 0:
            return
        def idx_copies(g):
            b = g % 2
            rows = pl.ds(base + g * GROUP, GROUP)
            return (pltpu.make_async_copy(src_hbm.at[rows], sidx.at[b], semi),
                    pltpu.make_async_copy(dst_hbm.at[rows], didx.at[b], semi))

        for cp in idx_copies(0):
            cp.start()
        for g in range(ngroup):
            for cp in idx_copies(g):
                cp.wait()
            if g + 1 < ngroup:
                for cp in idx_copies(g + 1):
                    cp.start()
            sv = sidx.at[g % 2]
            dv = didx.at[g % 2]
            for b in range(NB):
                pltpu.async_copy(hp_hbm.at[sv.at[b]], bufs[b], sems[b])

            def body(p, _, sv=sv, dv=dv):
                i = NB * p
                for b in range(NB):
                    pltpu.make_async_copy(
                        hp_hbm.at[sv.at[i + b]], bufs[b], sems[b]).wait()
                    pltpu.sync_copy(bufs[b], acc.at[dv.at[i + b]], add=True)
                    pltpu.async_copy(
                        hp_hbm.at[sv.at[i + NB + b]], bufs[b], sems[b])
                return 0

            lax.fori_loop(0, GROUP // NB - 1, body, 0)
            for b in range(NB):
                i = GROUP - NB + b
                pltpu.make_async_copy(
                    hp_hbm.at[sv.at[i]], bufs[b], sems[b]).wait()
                pltpu.sync_copy(bufs[b], acc.at[dv.at[i]], add=True)

    @pl.when(c == 0)
    def _():
        pipeline(s * CH0, CH0 // GROUP)

    @pl.when(c == 1)
    def _():
        pipeline(16 * CH0 + s * CH1, CH1 // GROUP)

    plsc.subcore_barrier()
    pltpu.sync_copy(acc.at[pl.ds(s * GZ, GZ)],
                    out_hbm.at[pl.ds(c * GS_ROWS + s * GZ, GZ)])


def _dis(c0_ref, c1_ref):
    deg = c0_ref[0][:, 0:1] + c1_ref[0][:, 0:1] + 1.0
    return lax.rsqrt(deg)


def _dense1_body(c0_ref, c1_ref, x_ref, w_ref, o_ref):
    xw = jnp.dot(x_ref[...], w_ref[...], preferred_element_type=jnp.float32,
                 precision=lax.Precision.HIGHEST)
    o_ref[...] = xw * _dis(c0_ref, c1_ref)


def _dense2_body(c0_ref, c1_ref, s0_ref, s1_ref, hp_ref, b_ref, w_ref, o_ref):
    dis = _dis(c0_ref, c1_ref)
    h1 = jnp.maximum(
        dis * (s0_ref[0] + s1_ref[0] + hp_ref[...]) + b_ref[...], 0.0)
    o_ref[...] = dis * jnp.dot(h1, w_ref[...], preferred_element_type=jnp.float32,
                               precision=lax.Precision.HIGHEST)


def _dense3_body(c0_ref, c1_ref, s0_ref, s1_ref, hp_ref, b_ref, h_ref, g_ref):
    dis = _dis(c0_ref, c1_ref)
    h2 = dis * (s0_ref[0] + s1_ref[0] + hp_ref[...]) + b_ref[...]
    h_ref[...] = h2

    @pl.when(pl.program_id(0) == 0)
    def _():
        g_ref[...] = jnp.zeros_like(g_ref)

    g_ref[...] += jnp.sum(h2, axis=0, keepdims=True) * (1.0 / N)


_row_spec = pl.BlockSpec((RBLK, D), lambda i: (i, 0))
_c0_spec = pl.BlockSpec((1, RBLK, 8), lambda i: (0, i, 0))
_c1_spec = pl.BlockSpec((1, RBLK, 8), lambda i: (1, i, 0))
_s0_spec = pl.BlockSpec((1, RBLK, D), lambda i: (0, i, 0))
_s1_spec = pl.BlockSpec((1, RBLK, D), lambda i: (1, i, 0))
_full_spec = pl.BlockSpec((D, D), lambda i: (0, 0))
_b_spec = pl.BlockSpec((1, D), lambda i: (0, 0))

_dense1 = pl.pallas_call(
    _dense1_body, grid=(N // RBLK,),
    in_specs=[_c0_spec, _c1_spec, _row_spec, _full_spec],
    out_specs=_row_spec,
    out_shape=jax.ShapeDtypeStruct((N, D), jnp.float32))

_dense2 = pl.pallas_call(
    _dense2_body, grid=(N // RBLK,),
    in_specs=[_c0_spec, _c1_spec, _s0_spec, _s1_spec, _row_spec, _b_spec,
              _full_spec],
    out_specs=_row_spec,
    out_shape=jax.ShapeDtypeStruct((N, D), jnp.float32))

_dense3 = pl.pallas_call(
    _dense3_body, grid=(N // RBLK,),
    in_specs=[_c0_spec, _c1_spec, _s0_spec, _s1_spec, _row_spec, _b_spec],
    out_specs=[_row_spec, pl.BlockSpec((1, D), lambda i: (0, 0))],
    out_shape=[jax.ShapeDtypeStruct((N, D), jnp.float32),
               jax.ShapeDtypeStruct((1, D), jnp.float32)])


def kernel(x, edge_index, W1, b1, W2, b2):
    src = edge_index[0].astype(jnp.int32)
    dst = edge_index[1].astype(jnp.int32)
    e = src.shape[0]
    pad = CAP - e
    pad_idx = jnp.arange(pad, dtype=jnp.int32)
    src_p = jnp.concatenate([src, pad_idx % N]).reshape(NW * NCHUNK, CHUNK)
    dst_full = jnp.concatenate([dst, N + pad_idx % (GS_ROWS - N)])
    dst_p = dst_full.reshape(NW * NCHUNK, CHUNK)
    dst_p_cnt = dst_full.reshape(NW * CNT_NCHUNK, CNT_CHUNK)
    const = jnp.stack([jnp.ones((CNT_CHUNK, 8), jnp.float32),
                       jnp.zeros((CNT_CHUNK, 8), jnp.float32)])
    zeros_rows = jnp.zeros((CHUNK, D), jnp.float32)

    cnt8 = _cnt_call(dst_p_cnt, const).reshape(2, ACC_ROWS, 8)

    hp1 = _dense1(cnt8, cnt8, x.astype(jnp.float32), W1)
    s1 = _gs_call(hp1, src_p, dst_p, zeros_rows).reshape(2, GS_ROWS, D)
    hp2 = _dense2(cnt8, cnt8, s1, s1, hp1, b1.reshape(1, D), W2)
    s2 = _gs_call(hp2, src_p, dst_p, zeros_rows).reshape(2, GS_ROWS, D)
    h2, g = _dense3(cnt8, cnt8, s2, s2, hp2, b2.reshape(1, D))
    return h2, g
